# Initial kernel scaffold; baseline (speedup 1.0000x reference)
import jax
import jax.numpy as jnp
from jax.experimental import pallas as pl


def kernel(features, rand_proj, codebook):
    raise NotImplementedError("write your pallas kernel here")


# speedup vs baseline: 1.0623x; 1.0623x over previous
import jax
import jax.numpy as jnp
from jax.experimental import pallas as pl

_K = 1000
_KP = 1024
_D = 80
_TB = 1024


def _prep_kernel(cb_ref, cbn_ref):
    cb = cb_ref[...]
    n = jnp.sqrt(jnp.sum(cb * cb, axis=-1, keepdims=True))
    cbn_ref[...] = (cb / jnp.clip(n, 1e-12, None)).astype(jnp.bfloat16)


def _label_kernel(x_ref, rp_ref, cbn_ref, o_ref):
    x = x_ref[...]
    mu = jnp.mean(x, axis=-1, keepdims=True)
    xc = x - mu
    var = jnp.mean(xc * xc, axis=-1, keepdims=True)
    xn = xc / jnp.sqrt(var + 1e-5)
    proj = jax.lax.dot_general(
        xn.astype(jnp.bfloat16), rp_ref[...],
        (((1,), (0,)), ((), ())), preferred_element_type=jnp.float32)
    pn = proj / jnp.clip(
        jnp.sqrt(jnp.sum(proj * proj, axis=-1, keepdims=True)), 1e-12, None)
    sim = jax.lax.dot_general(
        pn.astype(jnp.bfloat16), cbn_ref[...],
        (((1,), (1,)), ((), ())), preferred_element_type=jnp.float32)
    col = jax.lax.broadcasted_iota(jnp.int32, sim.shape, 1)
    sim = jnp.where(col < _K, sim, -jnp.inf)
    o_ref[0, 0, :] = jnp.argmax(sim, axis=-1).astype(jnp.int32)


def kernel(features, rand_proj, codebook):
    B, T, D = features.shape
    K, E = codebook.shape
    cb_pad = jnp.pad(codebook, ((0, _KP - K), (0, 0)))
    cbn = pl.pallas_call(
        _prep_kernel,
        out_shape=jax.ShapeDtypeStruct((_KP, E), jnp.bfloat16),
    )(cb_pad)
    rp_bf = rand_proj.astype(jnp.bfloat16)
    N = B * T
    nb = N // _TB
    xf = features.reshape(N, D)
    out = pl.pallas_call(
        _label_kernel,
        grid=(nb,),
        in_specs=[pl.BlockSpec((_TB, D), lambda i: (i, 0)),
                  pl.BlockSpec((D, E), lambda i: (0, 0)),
                  pl.BlockSpec((_KP, E), lambda i: (0, 0))],
        out_specs=pl.BlockSpec((1, 1, _TB), lambda i: (i, 0, 0)),
        out_shape=jax.ShapeDtypeStruct((nb, 1, _TB), jnp.int32),
    )(xf, rp_bf, cbn)
    return out.reshape(B, T)
